# Initial kernel scaffold; baseline (speedup 1.0000x reference)
import jax
import jax.numpy as jnp
from jax.experimental import pallas as pl


def kernel(boxes, scores):
    raise NotImplementedError("write your pallas kernel here")


# speedup vs baseline: 4.4810x; 4.4810x over previous
import jax
import jax.numpy as jnp
from jax.experimental import pallas as pl

_N = 20000
_K = 80
_SCORE_THRESH = 0.05
_NMS_THRESH = 0.5
_PRE_NMS = 1000
_TOPK = 100
_IMG_W = 1333.0
_IMG_H = 800.0
_PAD = 1024
_OUTROWS = 128
_ROWS_BLK = 2000


def _thresh_clip_kernel(scores_ref, boxes_ref, fgm_ref, cbox_ref):
    s = scores_ref[...]
    fg = s[:, :_K]
    fgm_ref[...] = jnp.where(fg > _SCORE_THRESH, fg, -jnp.inf)
    b = boxes_ref[...]
    col = jax.lax.broadcasted_iota(jnp.int32, b.shape, 1)
    bound = jnp.where(col % 2 == 0, _IMG_W, _IMG_H)
    cbox_ref[...] = jnp.minimum(jnp.maximum(b, 0.0), bound)


def _nms_kernel(x1_ref, y1_ref, x2_ref, y2_ref, vals_ref, cls_ref,
                onehot_ref, outv_ref):
    off = (_IMG_W + _IMG_H + 1.0) * cls_ref[...]
    x1 = x1_ref[...] + off
    y1 = y1_ref[...] + off
    x2 = x2_ref[...] + off
    y2 = y2_ref[...] + off
    vals = vals_ref[...]
    col = jax.lax.broadcasted_iota(jnp.int32, (1, _PAD), 1)
    area = jnp.maximum(x2 - x1, 0.0) * jnp.maximum(y2 - y1, 0.0)
    keep0 = jnp.where(vals != -jnp.inf, 1.0, 0.0)

    def nms_body(i, keep):
        sel = jnp.where(col == i, 1.0, 0.0)
        xi1 = jnp.sum(sel * x1)
        yi1 = jnp.sum(sel * y1)
        xi2 = jnp.sum(sel * x2)
        yi2 = jnp.sum(sel * y2)
        ki = jnp.sum(sel * keep)
        ai = jnp.maximum(xi2 - xi1, 0.0) * jnp.maximum(yi2 - yi1, 0.0)
        iw = jnp.maximum(jnp.minimum(xi2, x2) - jnp.maximum(xi1, x1), 0.0)
        ih = jnp.maximum(jnp.minimum(yi2, y2) - jnp.maximum(yi1, y1), 0.0)
        inter = iw * ih
        iou = inter / jnp.maximum(ai + area - inter, 1e-9)
        sup = jnp.where((iou > _NMS_THRESH) & (col > i), 1.0, 0.0) * ki
        return keep * (1.0 - sup)

    keep = jax.lax.fori_loop(0, _PRE_NMS, nms_body, keep0)
    kept = jnp.where(keep > 0.0, vals, -jnp.inf)
    col_out = jax.lax.broadcasted_iota(jnp.int32, (1, _OUTROWS), 1)

    def sel_body(j, carry):
        kv, outv = carry
        m = jnp.max(kv)
        gate = m != -jnp.inf
        ismax = (kv == m) & gate
        pos = jnp.min(jnp.where(ismax, col, 2 * _PAD))
        oh = jnp.where((col == pos) & gate, 1.0, 0.0)
        onehot_ref[pl.ds(j, 1), :] = oh
        outv = jnp.where(col_out == j, m, outv)
        kv = jnp.where(col == pos, -jnp.inf, kv)
        return kv, outv

    outv0 = jnp.full((1, _OUTROWS), -jnp.inf, jnp.float32)
    _, outv = jax.lax.fori_loop(0, _OUTROWS, sel_body, (kept, outv0))
    outv_ref[...] = outv


def kernel(boxes, scores):
    nblk = _N // _ROWS_BLK
    fgm, cboxes = pl.pallas_call(
        _thresh_clip_kernel,
        grid=(nblk,),
        in_specs=[
            pl.BlockSpec((_ROWS_BLK, _K + 1), lambda i: (i, 0)),
            pl.BlockSpec((_ROWS_BLK, 4), lambda i: (i, 0)),
        ],
        out_specs=[
            pl.BlockSpec((_ROWS_BLK, _K), lambda i: (i, 0)),
            pl.BlockSpec((_ROWS_BLK, 4), lambda i: (i, 0)),
        ],
        out_shape=[
            jax.ShapeDtypeStruct((_N, _K), jnp.float32),
            jax.ShapeDtypeStruct((_N, 4), jnp.float32),
        ],
    )(scores, boxes)

    flat = fgm.reshape(-1)
    top_vals, top_idx = jax.lax.top_k(flat, _PRE_NMS)
    box_idx = top_idx // _K
    cls = top_idx % _K
    cand = cboxes[box_idx]

    npad = _PAD - _PRE_NMS
    vals_p = jnp.concatenate(
        [top_vals, jnp.full((npad,), -jnp.inf, jnp.float32)]).reshape(1, _PAD)
    cand_p = jnp.concatenate(
        [cand, jnp.zeros((npad, 4), jnp.float32)], axis=0)
    cls_p = jnp.concatenate(
        [cls.astype(jnp.float32), jnp.zeros((npad,), jnp.float32)]
    ).reshape(1, _PAD)
    x1 = cand_p[:, 0].reshape(1, _PAD)
    y1 = cand_p[:, 1].reshape(1, _PAD)
    x2 = cand_p[:, 2].reshape(1, _PAD)
    y2 = cand_p[:, 3].reshape(1, _PAD)

    onehot, outv = pl.pallas_call(
        _nms_kernel,
        out_shape=[
            jax.ShapeDtypeStruct((_OUTROWS, _PAD), jnp.float32),
            jax.ShapeDtypeStruct((1, _OUTROWS), jnp.float32),
        ],
    )(x1, y1, x2, y2, vals_p, cls_p)

    oh = onehot[:_TOPK]
    outv_t = outv[0, :_TOPK]
    out_valid = outv_t != -jnp.inf
    sel_vals = jnp.where(jnp.isfinite(vals_p), vals_p, 0.0).reshape(_PAD, 1)
    out = oh @ jnp.concatenate([cand_p, sel_vals], axis=1)
    out_cls = jnp.where(
        out_valid,
        jnp.round(oh @ cls_p.reshape(_PAD)).astype(jnp.int32),
        -1,
    )
    return out, out_cls
